# baseline (speedup 1.0000x reference)
import jax
import jax.numpy as jnp
from jax.experimental import pallas as pl
from jax.experimental.pallas import tpu as pltpu
from jax.experimental.pallas import tpu_sc as plsc

V = 100000
D = 64
H = 128
B = 1024
VT = 2048
NV = (V + VT - 1) // VT
RB = 512
NB = B // RB
GW = 128


def _sc_gather(emb2, idx):
    mesh = plsc.VectorSubcoreMesh(core_axis_name="c", subcore_axis_name="s")
    nw = 32
    bpw = B // nw

    @pl.kernel(
        out_type=jax.ShapeDtypeStruct((B, 2 * D), emb2.dtype),
        mesh=mesh,
        scratch_types=[
            pltpu.VMEM((bpw,), jnp.int32),
            pltpu.VMEM((bpw, 2 * D), jnp.float32),
            pltpu.SemaphoreType.DMA,
        ],
    )
    def k(emb_hbm, idx_hbm, out_hbm, idx_v, rows_v, sem):
        wid = jax.lax.axis_index("s") * 2 + jax.lax.axis_index("c")
        base = wid * bpw
        pltpu.sync_copy(idx_hbm.at[pl.ds(base, bpw)], idx_v)
        pltpu.async_copy(emb_hbm.at[idx_v], rows_v, sem).wait()
        pltpu.sync_copy(rows_v, out_hbm.at[pl.ds(base, bpw)])

    return k(emb2, idx)


def _hidden_body(rows_ref, par_ref, wp_ref, bp_ref, h_ref):
    rows = rows_ref[...]
    g = jnp.where(par_ref[...] == 1, rows[:, D:], rows[:, :D])
    acc = jnp.dot(g, wp_ref[...].T, preferred_element_type=jnp.float32)
    h_ref[...] = jnp.maximum(acc + bp_ref[...], 0.0)


def _lse_body(h_ref, w_ref, b_ref, lse_ref, m_ref, s_ref):
    j = pl.program_id(1)

    @pl.when(j == 0)
    def _():
        m_ref[...] = jnp.full_like(m_ref, -jnp.inf)
        s_ref[...] = jnp.zeros_like(s_ref)

    logits = jnp.dot(h_ref[...], w_ref[...].T,
                     preferred_element_type=jnp.float32) + b_ref[...]
    col = j * VT + jax.lax.broadcasted_iota(jnp.int32, logits.shape, 1)
    logits = jnp.where(col < V, logits, -jnp.inf)
    tmax = jnp.max(logits, axis=1, keepdims=True)
    m_old = m_ref[...]
    m_new = jnp.maximum(m_old, tmax)
    s_ref[...] = (s_ref[...] * jnp.exp(m_old - m_new)
                  + jnp.sum(jnp.exp(logits - m_new), axis=1, keepdims=True))
    m_ref[...] = m_new

    @pl.when(j == NV - 1)
    def _():
        lse_ref[...] = jnp.broadcast_to(m_ref[...] + jnp.log(s_ref[...]),
                                        lse_ref.shape)


def _out_body(h_ref, w_ref, b_ref, lse_ref, o_ref):
    logits = jnp.dot(h_ref[...], w_ref[...].T,
                     preferred_element_type=jnp.float32) + b_ref[...]
    o_ref[...] = logits - lse_ref[...][:, 0:1]


def kernel(inputs, emb, W_proj, b_proj, W_out, b_out):
    idx = inputs.astype(jnp.int32)
    b_proj2 = b_proj.reshape(1, H)
    b_out2 = b_out.reshape(1, V)

    emb2 = emb.reshape(V // 2, 2 * D)
    rows = _sc_gather(emb2, idx >> 1)
    parity = (idx & 1).reshape(B, 1)

    h = pl.pallas_call(
        _hidden_body,
        out_shape=jax.ShapeDtypeStruct((B, H), jnp.float32),
    )(rows, parity, W_proj, b_proj2)

    lse = pl.pallas_call(
        _lse_body,
        grid=(NB, NV),
        in_specs=[
            pl.BlockSpec((RB, H), lambda i, j: (i, 0)),
            pl.BlockSpec((VT, H), lambda i, j: (j, 0)),
            pl.BlockSpec((1, VT), lambda i, j: (0, j)),
        ],
        out_specs=pl.BlockSpec((RB, 128), lambda i, j: (i, 0)),
        out_shape=jax.ShapeDtypeStruct((B, 128), jnp.float32),
        scratch_shapes=[
            pltpu.VMEM((RB, 1), jnp.float32),
            pltpu.VMEM((RB, 1), jnp.float32),
        ],
        compiler_params=pltpu.CompilerParams(
            dimension_semantics=("parallel", "arbitrary")),
    )(h, W_out, b_out2)

    out = pl.pallas_call(
        _out_body,
        grid=(NV,),
        in_specs=[
            pl.BlockSpec((B, H), lambda j: (0, 0)),
            pl.BlockSpec((VT, H), lambda j: (j, 0)),
            pl.BlockSpec((1, VT), lambda j: (0, j)),
            pl.BlockSpec((B, 128), lambda j: (0, 0)),
        ],
        out_specs=pl.BlockSpec((B, VT), lambda j: (0, j)),
        out_shape=jax.ShapeDtypeStruct((B, V), jnp.float32),
        compiler_params=pltpu.CompilerParams(
            dimension_semantics=("parallel",)),
    )(h, W_out, b_out2, lse)

    return out
